# baseline (speedup 1.0000x reference)
import functools
import math

import jax
import jax.numpy as jnp
from jax import lax
from jax.experimental import pallas as pl
from jax.experimental.pallas import tpu as pltpu
from jax.experimental.pallas import tpu_sc as plsc

D_MODEL = 64
SCALE = math.sqrt(D_MODEL)
NUM_CORES = 2
NUM_SUBCORES = 16
NUM_WORKERS = NUM_CORES * NUM_SUBCORES
CHUNK = 128
LANES = 16


def _emb_body(n_chunks, b_per_w, x_hbm, table_hbm, out_hbm, idx_v, rows_v, sem):
    wid = lax.axis_index("s") * NUM_CORES + lax.axis_index("c")
    base = wid * b_per_w
    pltpu.sync_copy(x_hbm.at[pl.ds(base, b_per_w)], idx_v)

    def chunk_body(g, _):
        off = g * CHUNK
        pltpu.async_copy(
            table_hbm.at[idx_v.at[pl.ds(off, CHUNK)]], rows_v, sem
        ).wait()

        def row_body(i, _):
            for j in range(D_MODEL // LANES):
                s = pl.ds(j * LANES, LANES)
                rows_v[i, s] = rows_v[i, s] * SCALE
            return 0

        lax.fori_loop(0, CHUNK, row_body, 0)
        pltpu.sync_copy(rows_v, out_hbm.at[pl.ds(base + off, CHUNK)])
        return 0

    lax.fori_loop(0, n_chunks, chunk_body, 0)


def kernel(x, table):
    b_total = x.size
    assert b_total % (NUM_WORKERS * CHUNK) == 0
    b_per_w = b_total // NUM_WORKERS
    n_chunks = b_per_w // CHUNK
    xf = x.reshape(b_total)

    mesh = plsc.VectorSubcoreMesh(
        core_axis_name="c",
        subcore_axis_name="s",
        num_cores=NUM_CORES,
        num_subcores=NUM_SUBCORES,
    )
    grid_fn = pl.kernel(
        functools.partial(_emb_body, n_chunks, b_per_w),
        out_type=jax.ShapeDtypeStruct((b_total, D_MODEL), jnp.float32),
        mesh=mesh,
        scratch_types=[
            pltpu.VMEM((b_per_w,), jnp.int32),
            pltpu.VMEM((CHUNK, D_MODEL), jnp.float32),
            pltpu.SemaphoreType.DMA,
        ],
        compiler_params=pltpu.CompilerParams(use_tc_tiling_on_sc=False),
    )
    out = grid_fn(xf, table)
    return out.reshape(x.shape + (D_MODEL,))


# speedup vs baseline: 1.2097x; 1.2097x over previous
import functools
import math

import jax
import jax.numpy as jnp
from jax import lax
from jax.experimental import pallas as pl
from jax.experimental.pallas import tpu as pltpu
from jax.experimental.pallas import tpu_sc as plsc

D_MODEL = 64
SCALE = math.sqrt(D_MODEL)
NUM_CORES = 2
NUM_SUBCORES = 16
NUM_WORKERS = NUM_CORES * NUM_SUBCORES
CHUNK = 128
NBUF = 4
LANES = 16


def _emb_body(n_chunks, b_per_w, x_hbm, table_hbm, out_hbm, idx_v,
              g0, g1, g2, g3, s0, s1, s2, s3,
              gsem0, gsem1, gsem2, gsem3, ssem0, ssem1, ssem2, ssem3):
    gbuf = (g0, g1, g2, g3)
    sbuf = (s0, s1, s2, s3)
    gsem = (gsem0, gsem1, gsem2, gsem3)
    ssem = (ssem0, ssem1, ssem2, ssem3)

    wid = lax.axis_index("s") * NUM_CORES + lax.axis_index("c")
    base = wid * b_per_w
    pltpu.sync_copy(x_hbm.at[pl.ds(base, b_per_w)], idx_v)

    def start_gather(c, b):
        pltpu.async_copy(
            table_hbm.at[idx_v.at[pl.ds(c * CHUNK, CHUNK)]], gbuf[b], gsem[b]
        )

    def wait_gather(c, b):
        pltpu.make_async_copy(
            table_hbm.at[idx_v.at[pl.ds(c * CHUNK, CHUNK)]], gbuf[b], gsem[b]
        ).wait()

    def start_store(c, b):
        pltpu.async_copy(
            sbuf[b], out_hbm.at[pl.ds(base + c * CHUNK, CHUNK)], ssem[b]
        )

    def wait_store(c, b):
        pltpu.make_async_copy(
            sbuf[b], out_hbm.at[pl.ds(base + c * CHUNK, CHUNK)], ssem[b]
        ).wait()

    def scale(b):
        @plsc.parallel_loop(0, CHUNK, unroll=4)
        def _(i):
            for j in range(D_MODEL // LANES):
                s = pl.ds(j * LANES, LANES)
                sbuf[b][i, s] = gbuf[b][i, s] * SCALE

    for b in range(NBUF):
        start_gather(b, b)

    @pl.loop(0, n_chunks - NBUF, step=NBUF)
    def _(g):
        for b in range(NBUF):
            c = g + b
            wait_gather(c, b)
            @pl.when(c >= NBUF)
            def _():
                wait_store(c - NBUF, b)
            scale(b)
            start_store(c, b)
            start_gather(c + NBUF, b)

    for b in range(NBUF):
        c = n_chunks - NBUF + b
        wait_gather(c, b)
        wait_store(c - NBUF, b)
        scale(b)
        start_store(c, b)
    for b in range(NBUF):
        wait_store(n_chunks - NBUF + b, b)


def kernel(x, table):
    b_total = x.size
    assert b_total % (NUM_WORKERS * CHUNK) == 0
    b_per_w = b_total // NUM_WORKERS
    n_chunks = b_per_w // CHUNK
    assert (n_chunks - NBUF) % NBUF == 0
    xf = x.reshape(b_total)

    mesh = plsc.VectorSubcoreMesh(
        core_axis_name="c",
        subcore_axis_name="s",
        num_cores=NUM_CORES,
        num_subcores=NUM_SUBCORES,
    )
    grid_fn = pl.kernel(
        functools.partial(_emb_body, n_chunks, b_per_w),
        out_type=jax.ShapeDtypeStruct((b_total, D_MODEL), jnp.float32),
        mesh=mesh,
        scratch_types=(
            [pltpu.VMEM((b_per_w,), jnp.int32)]
            + [pltpu.VMEM((CHUNK, D_MODEL), jnp.float32)] * (2 * NBUF)
            + [pltpu.SemaphoreType.DMA] * (2 * NBUF)
        ),
        compiler_params=pltpu.CompilerParams(use_tc_tiling_on_sc=False),
    )
    out = grid_fn(xf, table)
    return out.reshape(x.shape + (D_MODEL,))
